# baseline (speedup 1.0000x reference)
import functools

import jax
import jax.numpy as jnp
from jax import lax
from jax.experimental import pallas as pl
from jax.experimental.pallas import tpu as pltpu
from jax.experimental.pallas import tpu_sc as plsc

NUM_PAGES, PAGE_SIZE, KV_DIM = 2048, 16, 1024
MAX_SEQS, MAX_LEN = 128, 8192
PAGES_PER_SEQ = MAX_LEN // PAGE_SIZE
B = 64

GRID = 16
CROWS = NUM_PAGES // GRID

_SC_INFO = plsc.get_sparse_core_info()
NC, NS, L = _SC_INFO.num_cores, _SC_INFO.num_subcores, _SC_INFO.num_lanes
NW = NC * NS
RPW = MAX_SEQS // NW


def _sc_body(tok_hbm, kv_hbm, sl_hbm, par_hbm, chi_hbm,
             out_tok, out_kv, out_sl,
             tsel_v, par_v, chi_v, idx_v, sl_v, slo_v, tbuf, kbuf):
    wid = lax.axis_index("s") * NC + lax.axis_index("c")
    lanes = lax.iota(jnp.int32, L)

    pltpu.sync_copy(par_hbm, par_v)
    pltpu.sync_copy(chi_hbm, chi_v)
    pltpu.sync_copy(sl_hbm, sl_v)

    for c in range(MAX_SEQS // L):
        tsel_v[pl.ds(c * L, L)] = lanes + c * L
    for c in range(B // L):
        chi = chi_v[pl.ds(c * L, L)]
        par = par_v[pl.ds(c * L, L)]
        plsc.store_scatter(tsel_v, [chi], par)

    rowids = jnp.minimum(wid * RPW + lanes, MAX_SEQS - 1)
    vals = plsc.load_gather(tsel_v, [rowids])
    plsc.store_scatter(idx_v, [lanes], vals, mask=lanes < RPW)

    pltpu.sync_copy(tok_hbm.at[idx_v], tbuf)
    pltpu.sync_copy(kv_hbm.at[idx_v], kbuf)
    pltpu.sync_copy(tbuf, out_tok.at[pl.ds(wid * RPW, RPW)])
    pltpu.sync_copy(kbuf, out_kv.at[pl.ds(wid * RPW, RPW)])

    @pl.when(wid < MAX_SEQS // L)
    def _():
        tchunk = tsel_v[pl.ds(wid * L, L)]
        slo_v[...] = plsc.load_gather(sl_v, [tchunk])
        pltpu.sync_copy(slo_v, out_sl.at[pl.ds(wid * L, L)])


def _sc_clone(tokens, kv_pages, seq_lens, parent_ids, child_ids):
    mesh = plsc.VectorSubcoreMesh(core_axis_name="c", subcore_axis_name="s")
    return pl.kernel(
        _sc_body,
        mesh=mesh,
        compiler_params=pltpu.CompilerParams(needs_layout_passes=False),
        out_type=(
            jax.ShapeDtypeStruct(tokens.shape, tokens.dtype),
            jax.ShapeDtypeStruct(kv_pages.shape, kv_pages.dtype),
            jax.ShapeDtypeStruct(seq_lens.shape, seq_lens.dtype),
        ),
        scratch_types=[
            pltpu.VMEM((MAX_SEQS,), jnp.int32),
            pltpu.VMEM((B,), jnp.int32),
            pltpu.VMEM((B,), jnp.int32),
            pltpu.VMEM((RPW,), jnp.int32),
            pltpu.VMEM((MAX_SEQS,), jnp.int32),
            pltpu.VMEM((L,), jnp.int32),
            pltpu.VMEM((RPW, MAX_LEN), jnp.int32),
            pltpu.VMEM((RPW, PAGES_PER_SEQ), jnp.int32),
        ],
    )(tokens, kv_pages, seq_lens, parent_ids, child_ids)


def _tc_body(cin, cany, psref, pdref, cout, cfix, csem):
    i = pl.program_id(0)

    def cdma(j):
        return pltpu.make_async_copy(cany.at[psref[j]], cfix.at[j], csem)

    @pl.when(i == 0)
    def _():
        def issue(j, _):
            cdma(j).start()
            return 0

        jax.lax.fori_loop(0, B, issue, 0)

        def drain(j, _):
            cdma(j).wait()
            return 0

        jax.lax.fori_loop(0, B, drain, 0)

    cout[...] = cin[...]

    def fixc(j, _):
        dst = pdref[j]

        @pl.when(dst // CROWS == i)
        def _():
            cout[pl.ds(dst % CROWS, 1)] = cfix[pl.ds(j, 1)]

        return 0

    jax.lax.fori_loop(0, B, fixc, 0)


def _tc_cache(cache, page_src, page_dst):
    smem = functools.partial(pl.BlockSpec, memory_space=pltpu.SMEM)
    any_ = functools.partial(pl.BlockSpec, memory_space=pl.ANY)
    cblk = pl.BlockSpec((CROWS, PAGE_SIZE, KV_DIM), lambda i: (i, 0, 0))
    return pl.pallas_call(
        _tc_body,
        grid=(GRID,),
        in_specs=[cblk, any_(), smem(), smem()],
        out_specs=cblk,
        out_shape=jax.ShapeDtypeStruct(cache.shape, cache.dtype),
        scratch_shapes=[
            pltpu.VMEM((B, PAGE_SIZE, KV_DIM), cache.dtype),
            pltpu.SemaphoreType.DMA,
        ],
    )(cache, cache, page_src, page_dst)


def kernel(cache, tokens, kv_pages, seq_lens, parent_ids, child_ids, page_src, page_dst):
    new_tokens, new_kv, new_sl = _sc_clone(tokens, kv_pages, seq_lens,
                                           parent_ids, child_ids)
    new_cache = _tc_cache(cache, page_src, page_dst)
    return new_cache, new_tokens, new_kv, new_sl


# speedup vs baseline: 1.0010x; 1.0010x over previous
import functools

import jax
import jax.numpy as jnp
from jax import lax
from jax.experimental import pallas as pl
from jax.experimental.pallas import tpu as pltpu
from jax.experimental.pallas import tpu_sc as plsc

NUM_PAGES, PAGE_SIZE, KV_DIM = 2048, 16, 1024
MAX_SEQS, MAX_LEN = 128, 8192
PAGES_PER_SEQ = MAX_LEN // PAGE_SIZE
B = 64

GRID = 16
CROWS = NUM_PAGES // GRID

_SC_INFO = plsc.get_sparse_core_info()
NC, NS, L = _SC_INFO.num_cores, _SC_INFO.num_subcores, _SC_INFO.num_lanes
NW = NC * NS
RPW = MAX_SEQS // NW


def _sc_body(tok_hbm, kv_hbm, sl_hbm, par_hbm, chi_hbm,
             out_tok, out_kv, out_sl,
             tsel_v, par_v, chi_v, idx_v, sl_v, slo_v, tbuf, kbuf):
    wid = lax.axis_index("s") * NC + lax.axis_index("c")
    lanes = lax.iota(jnp.int32, L)

    pltpu.sync_copy(par_hbm, par_v)
    pltpu.sync_copy(chi_hbm, chi_v)
    pltpu.sync_copy(sl_hbm, sl_v)

    for c in range(MAX_SEQS // L):
        tsel_v[pl.ds(c * L, L)] = lanes + c * L
    for c in range(B // L):
        chi = chi_v[pl.ds(c * L, L)]
        par = par_v[pl.ds(c * L, L)]
        plsc.store_scatter(tsel_v, [chi], par)

    rowids = jnp.minimum(wid * RPW + lanes, MAX_SEQS - 1)
    vals = plsc.load_gather(tsel_v, [rowids])
    plsc.store_scatter(idx_v, [lanes], vals, mask=lanes < RPW)

    pltpu.sync_copy(tok_hbm.at[idx_v], tbuf)
    pltpu.sync_copy(kv_hbm.at[idx_v], kbuf)
    pltpu.sync_copy(tbuf, out_tok.at[pl.ds(wid * RPW, RPW)])
    pltpu.sync_copy(kbuf, out_kv.at[pl.ds(wid * RPW, RPW)])

    @pl.when(wid < MAX_SEQS // L)
    def _():
        tchunk = tsel_v[pl.ds(wid * L, L)]
        slo_v[...] = plsc.load_gather(sl_v, [tchunk])
        pltpu.sync_copy(slo_v, out_sl.at[pl.ds(wid * L, L)])


def _sc_clone(tokens, kv_pages, seq_lens, parent_ids, child_ids):
    mesh = plsc.VectorSubcoreMesh(core_axis_name="c", subcore_axis_name="s")
    return pl.kernel(
        _sc_body,
        mesh=mesh,
        compiler_params=pltpu.CompilerParams(needs_layout_passes=False),
        out_type=(
            jax.ShapeDtypeStruct(tokens.shape, tokens.dtype),
            jax.ShapeDtypeStruct(kv_pages.shape, kv_pages.dtype),
            jax.ShapeDtypeStruct(seq_lens.shape, seq_lens.dtype),
        ),
        scratch_types=[
            pltpu.VMEM((MAX_SEQS,), jnp.int32),
            pltpu.VMEM((B,), jnp.int32),
            pltpu.VMEM((B,), jnp.int32),
            pltpu.VMEM((RPW,), jnp.int32),
            pltpu.VMEM((MAX_SEQS,), jnp.int32),
            pltpu.VMEM((L,), jnp.int32),
            pltpu.VMEM((RPW, MAX_LEN), jnp.int32),
            pltpu.VMEM((RPW, PAGES_PER_SEQ), jnp.int32),
        ],
    )(tokens, kv_pages, seq_lens, parent_ids, child_ids)


def _tc_body(cin, cany, psref, pdref, cout, cfix, csem):
    i = pl.program_id(0)

    def cdma(j):
        return pltpu.make_async_copy(cany.at[psref[j]], cfix.at[j], csem)

    @pl.when(i == 0)
    def _():
        def issue(j, _):
            cdma(j).start()
            return 0

        jax.lax.fori_loop(0, B, issue, 0)

        def drain(j, _):
            cdma(j).wait()
            return 0

        jax.lax.fori_loop(0, B, drain, 0)

    cout[...] = cin[...]

    def fixc(j, _):
        dst = pdref[j]

        @pl.when(dst // CROWS == i)
        def _():
            cout[pl.ds(dst % CROWS, 1)] = cfix[pl.ds(j, 1)]

        return 0

    jax.lax.fori_loop(0, B, fixc, 0)


def _tc_cache(cache, page_src, page_dst):
    smem = functools.partial(pl.BlockSpec, memory_space=pltpu.SMEM)
    any_ = functools.partial(pl.BlockSpec, memory_space=pl.ANY)
    cblk = pl.BlockSpec((CROWS, PAGE_SIZE, KV_DIM), lambda i: (i, 0, 0))
    return pl.pallas_call(
        _tc_body,
        grid=(GRID,),
        in_specs=[cblk, any_(), smem(), smem()],
        out_specs=cblk,
        out_shape=jax.ShapeDtypeStruct(cache.shape, cache.dtype),
        scratch_shapes=[
            pltpu.VMEM((B, PAGE_SIZE, KV_DIM), cache.dtype),
            pltpu.SemaphoreType.DMA,
        ],
    )(cache, cache, page_src, page_dst)


def kernel(cache, tokens, kv_pages, seq_lens, parent_ids, child_ids, page_src, page_dst):
    new_cache = _tc_cache(cache, page_src, page_dst)
    new_tokens, new_kv, new_sl = _sc_clone(tokens, kv_pages, seq_lens,
                                           parent_ids, child_ids)
    return new_cache, new_tokens, new_kv, new_sl


# speedup vs baseline: 1.0080x; 1.0070x over previous
import functools

import jax
import jax.numpy as jnp
from jax import lax
from jax.experimental import pallas as pl
from jax.experimental.pallas import tpu as pltpu
from jax.experimental.pallas import tpu_sc as plsc

NUM_PAGES, PAGE_SIZE, KV_DIM = 2048, 16, 1024
MAX_SEQS, MAX_LEN = 128, 8192
PAGES_PER_SEQ = MAX_LEN // PAGE_SIZE
B = 64

GRID = 16
CROWS = NUM_PAGES // GRID

_SC_INFO = plsc.get_sparse_core_info()
NC, NS, L = _SC_INFO.num_cores, _SC_INFO.num_subcores, _SC_INFO.num_lanes
NW = NC * NS
RPW = MAX_SEQS // NW


def _sc_body(tok_hbm, kv_hbm, sl_hbm, par_hbm, chi_hbm,
             out_tok, out_kv, out_sl,
             tsel_v, par_v, chi_v, idx_v, sl_v, slo_v, tbuf, kbuf, sem, gsem):
    wid = lax.axis_index("s") * NC + lax.axis_index("c")
    lanes = lax.iota(jnp.int32, L)

    d_par = pltpu.make_async_copy(par_hbm, par_v, sem)
    d_chi = pltpu.make_async_copy(chi_hbm, chi_v, sem)
    d_sl = pltpu.make_async_copy(sl_hbm, sl_v, sem)
    d_par.start()
    d_chi.start()
    d_sl.start()

    for c in range(MAX_SEQS // L):
        tsel_v[pl.ds(c * L, L)] = lanes + c * L
    d_par.wait()
    d_chi.wait()
    for c in range(B // L):
        chi = chi_v[pl.ds(c * L, L)]
        par = par_v[pl.ds(c * L, L)]
        plsc.store_scatter(tsel_v, [chi], par)

    rowids = jnp.minimum(wid * RPW + lanes, MAX_SEQS - 1)
    vals = plsc.load_gather(tsel_v, [rowids])
    plsc.store_scatter(idx_v, [lanes], vals, mask=lanes < RPW)

    g_tok = pltpu.make_async_copy(tok_hbm.at[idx_v], tbuf, gsem)
    g_kv = pltpu.make_async_copy(kv_hbm.at[idx_v], kbuf, gsem)
    g_tok.start()
    g_kv.start()

    @pl.when(wid < MAX_SEQS // L)
    def _():
        d_sl.wait()
        tchunk = tsel_v[pl.ds(wid * L, L)]
        slo_v[...] = plsc.load_gather(sl_v, [tchunk])
        pltpu.make_async_copy(slo_v, out_sl.at[pl.ds(wid * L, L)], sem).start()

    @pl.when(wid >= MAX_SEQS // L)
    def _():
        d_sl.wait()

    g_tok.wait()
    g_kv.wait()
    w_tok = pltpu.make_async_copy(tbuf, out_tok.at[pl.ds(wid * RPW, RPW)], gsem)
    w_kv = pltpu.make_async_copy(kbuf, out_kv.at[pl.ds(wid * RPW, RPW)], gsem)
    w_tok.start()
    w_kv.start()
    w_tok.wait()
    w_kv.wait()

    @pl.when(wid < MAX_SEQS // L)
    def _():
        pltpu.make_async_copy(slo_v, out_sl.at[pl.ds(wid * L, L)], sem).wait()


def _sc_clone(tokens, kv_pages, seq_lens, parent_ids, child_ids):
    mesh = plsc.VectorSubcoreMesh(core_axis_name="c", subcore_axis_name="s")
    return pl.kernel(
        _sc_body,
        mesh=mesh,
        compiler_params=pltpu.CompilerParams(needs_layout_passes=False),
        out_type=(
            jax.ShapeDtypeStruct(tokens.shape, tokens.dtype),
            jax.ShapeDtypeStruct(kv_pages.shape, kv_pages.dtype),
            jax.ShapeDtypeStruct(seq_lens.shape, seq_lens.dtype),
        ),
        scratch_types=[
            pltpu.VMEM((MAX_SEQS,), jnp.int32),
            pltpu.VMEM((B,), jnp.int32),
            pltpu.VMEM((B,), jnp.int32),
            pltpu.VMEM((RPW,), jnp.int32),
            pltpu.VMEM((MAX_SEQS,), jnp.int32),
            pltpu.VMEM((L,), jnp.int32),
            pltpu.VMEM((RPW, MAX_LEN), jnp.int32),
            pltpu.VMEM((RPW, PAGES_PER_SEQ), jnp.int32),
            pltpu.SemaphoreType.DMA,
            pltpu.SemaphoreType.DMA,
        ],
    )(tokens, kv_pages, seq_lens, parent_ids, child_ids)


def _tc_body(cin, cany, psref, pdref, cout, cfix, csem):
    i = pl.program_id(0)

    def cdma(j):
        return pltpu.make_async_copy(cany.at[psref[j]], cfix.at[j], csem)

    @pl.when(i == 0)
    def _():
        def issue(j, _):
            cdma(j).start()
            return 0

        jax.lax.fori_loop(0, B, issue, 0)

        def drain(j, _):
            cdma(j).wait()
            return 0

        jax.lax.fori_loop(0, B, drain, 0)

    cout[...] = cin[...]

    def fixc(j, _):
        dst = pdref[j]

        @pl.when(dst // CROWS == i)
        def _():
            cout[pl.ds(dst % CROWS, 1)] = cfix[pl.ds(j, 1)]

        return 0

    jax.lax.fori_loop(0, B, fixc, 0)


def _tc_cache(cache, page_src, page_dst):
    smem = functools.partial(pl.BlockSpec, memory_space=pltpu.SMEM)
    any_ = functools.partial(pl.BlockSpec, memory_space=pl.ANY)
    cblk = pl.BlockSpec((CROWS, PAGE_SIZE, KV_DIM), lambda i: (i, 0, 0))
    return pl.pallas_call(
        _tc_body,
        grid=(GRID,),
        in_specs=[cblk, any_(), smem(), smem()],
        out_specs=cblk,
        out_shape=jax.ShapeDtypeStruct(cache.shape, cache.dtype),
        scratch_shapes=[
            pltpu.VMEM((B, PAGE_SIZE, KV_DIM), cache.dtype),
            pltpu.SemaphoreType.DMA,
        ],
    )(cache, cache, page_src, page_dst)


def kernel(cache, tokens, kv_pages, seq_lens, parent_ids, child_ids, page_src, page_dst):
    new_cache = _tc_cache(cache, page_src, page_dst)
    new_tokens, new_kv, new_sl = _sc_clone(tokens, kv_pages, seq_lens,
                                           parent_ids, child_ids)
    return new_cache, new_tokens, new_kv, new_sl
